# Initial kernel scaffold; baseline (speedup 1.0000x reference)
import jax
import jax.numpy as jnp
from jax.experimental import pallas as pl


def kernel(dense_input, large_emb_inputs, small_emb_inputs, large_tables, small_tables, bw0, bb0, bw1, bb1, bw2, bb2, dcn_U, dcn_V, dcn_b, tw0, tb0, tw1, tb1, tw2, tb2, tw3, tb3):
    raise NotImplementedError("write your pallas kernel here")


# speedup vs baseline: 2.4964x; 2.4964x over previous
import functools

import jax
import jax.numpy as jnp
from jax import lax
from jax.experimental import pallas as pl
from jax.experimental.pallas import tpu as pltpu
from jax.experimental.pallas import tpu_sc as plsc

_B = 4096
_DIN = 13
_D = 32
_NL = 22
_VL = 100000
_NS = 4
_VS = 1000
_L = 20
_X0 = 896
_NW = 32
_RPW = _B * _NL // _NW
_CHUNK = 128
_NCHUNK = _RPW // _CHUNK
_BAGS_PW = _B // 8

_sc_mesh = plsc.VectorSubcoreMesh(core_axis_name="c", subcore_axis_name="s")


@functools.partial(
    pl.kernel,
    out_type=jax.ShapeDtypeStruct((_B * _NL, _D), jnp.float32),
    mesh=_sc_mesh,
    compiler_params=pltpu.CompilerParams(use_tc_tiling_on_sc=False),
    scratch_types=[
        pltpu.VMEM((_NCHUNK, _CHUNK), jnp.int32),
        pltpu.VMEM((_RPW, _D), jnp.float32),
        pltpu.SemaphoreType.DMA,
    ],
)
def _large_gather(idx_hbm, table_hbm, out_hbm, idx_v, rows_v, sem):
    wid = lax.axis_index("s") * 2 + lax.axis_index("c")
    pltpu.sync_copy(idx_hbm.at[wid], idx_v)
    handles = [
        pltpu.async_copy(
            table_hbm.at[idx_v.at[k]],
            rows_v.at[pl.ds(k * _CHUNK, _CHUNK)],
            sem,
        )
        for k in range(_NCHUNK)
    ]
    for h in handles:
        h.wait()
    pltpu.sync_copy(rows_v, out_hbm.at[pl.ds(wid * _RPW, _RPW)])


@functools.partial(
    pl.kernel,
    out_type=jax.ShapeDtypeStruct((_NW, _D, _BAGS_PW), jnp.float32),
    mesh=_sc_mesh,
    compiler_params=pltpu.CompilerParams(needs_layout_passes=False),
    scratch_types=[
        pltpu.VMEM((_VS * _D,), jnp.float32),
        pltpu.VMEM((_L, _BAGS_PW), jnp.int32),
        pltpu.VMEM((_D, _BAGS_PW), jnp.float32),
    ],
)
def _small_bagsum(idx_hbm, tables_hbm, out_hbm, table_v, idx_v, out_v):
    wid = lax.axis_index("s") * 2 + lax.axis_index("c")
    feat = wid // 8
    pltpu.sync_copy(tables_hbm.at[pl.ds(feat * (_VS * _D), _VS * _D)], table_v)
    pltpu.sync_copy(idx_hbm.at[wid], idx_v)

    def g_body(g, carry):
        accs = [jnp.zeros((16,), jnp.float32) for _ in range(_D)]
        for l in range(_L):
            ids = idx_v[l, pl.ds(g * 16, 16)]
            word0 = ids * _D
            for c in range(_D):
                accs[c] = accs[c] + plsc.load_gather(table_v, [word0 + c])
        for c in range(_D):
            out_v[c, pl.ds(g * 16, 16)] = accs[c]
        return carry

    lax.fori_loop(0, _BAGS_PW // 16, g_body, 0)
    pltpu.sync_copy(out_v, out_hbm.at[wid])


_BB = 512


def _tc_body(dense_ref, small_ref, large_ref,
             bw0_ref, bb0_ref, bw1_ref, bb1_ref, bw2_ref, bb2_ref,
             U_ref, V_ref, db_ref,
             tw0_ref, tb0_ref, tw1_ref, tb1_ref, tw2_ref, tb2_ref,
             tw3_ref, tb3_ref, out_ref):
    f32 = jnp.float32
    h = dense_ref[...]
    h = jnp.maximum(jnp.dot(h, bw0_ref[...], preferred_element_type=f32)
                    + bb0_ref[...], 0.0)
    h = jnp.maximum(jnp.dot(h, bw1_ref[...], preferred_element_type=f32)
                    + bb1_ref[...], 0.0)
    h = jnp.maximum(jnp.dot(h, bw2_ref[...], preferred_element_type=f32)
                    + bb2_ref[...], 0.0)
    x0 = jnp.concatenate(
        [h, small_ref[...], large_ref[...],
         jnp.zeros((h.shape[0], _X0 - 864), f32)], axis=1)
    xl = x0
    for i in range(3):
        t = jnp.dot(xl, U_ref[i], preferred_element_type=f32)
        t = jnp.dot(t, V_ref[i], preferred_element_type=f32) + db_ref[i]
        xl = x0 * t + xl
    h = jnp.maximum(jnp.dot(xl, tw0_ref[...], preferred_element_type=f32)
                    + tb0_ref[...], 0.0)
    h = jnp.maximum(jnp.dot(h, tw1_ref[...], preferred_element_type=f32)
                    + tb1_ref[...], 0.0)
    h = jnp.maximum(jnp.dot(h, tw2_ref[...], preferred_element_type=f32)
                    + tb2_ref[...], 0.0)
    out_ref[...] = jax.nn.sigmoid(
        jnp.dot(h, tw3_ref[...], preferred_element_type=f32) + tb3_ref[...])


def kernel(dense_input, large_emb_inputs, small_emb_inputs, large_tables,
           small_tables, bw0, bb0, bw1, bb1, bw2, bb2, dcn_U, dcn_V, dcn_b,
           tw0, tb0, tw1, tb1, tw2, tb2, tw3, tb3):
    f32 = jnp.float32
    li = large_emb_inputs.astype(jnp.int32)
    flat_idx = (li + jnp.arange(_NL, dtype=jnp.int32)[None, :] * _VL)
    flat_idx = flat_idx.reshape(_NW, _NCHUNK, _CHUNK)
    large_rows = _large_gather(flat_idx, large_tables.reshape(_NL * _VL, _D))
    large_cat = large_rows.reshape(_B, _NL * _D)

    si = small_emb_inputs.astype(jnp.int32)
    idx_t = si.reshape(8, _BAGS_PW, _NS, _L).transpose(2, 0, 3, 1)
    idx_t = idx_t.reshape(_NW, _L, _BAGS_PW)
    small_t = _small_bagsum(idx_t, small_tables.reshape(_NS * _VS * _D))
    small_cat = small_t.reshape(_NS, 8, _D, _BAGS_PW).transpose(1, 3, 0, 2)
    small_cat = small_cat.reshape(_B, _NS * _D)

    dense_pad = jnp.pad(dense_input, ((0, 0), (0, 128 - _DIN)))
    bw0p = jnp.pad(bw0, ((0, 128 - _DIN), (0, 0)))
    Up = jnp.pad(dcn_U, ((0, 0), (0, _X0 - 864), (0, 0)))
    Vp = jnp.pad(dcn_V, ((0, 0), (0, 0), (0, _X0 - 864)))
    dbp = jnp.pad(dcn_b, ((0, 0), (0, _X0 - 864)))
    tw0p = jnp.pad(tw0, ((0, _X0 - 864), (0, 0)))
    tw3p = jnp.pad(tw3, ((0, 0), (0, 127)))
    tb3p = jnp.pad(tb3, ((0, 127)))

    row2 = lambda v: v.reshape(1, -1)
    full = lambda *shape: pl.BlockSpec(shape, lambda i: (0,) * len(shape))
    batched = lambda w: pl.BlockSpec((_BB, w), lambda i: (i, 0))

    out = pl.pallas_call(
        _tc_body,
        grid=(_B // _BB,),
        in_specs=[
            batched(128), batched(_NS * _D), batched(_NL * _D),
            full(128, 512), full(1, 512), full(512, 256), full(1, 256),
            full(256, 32), full(1, 32),
            full(3, _X0, 128), full(3, 128, _X0), full(3, _X0),
            full(_X0, 1024), full(1, 1024), full(1024, 512), full(1, 512),
            full(512, 256), full(1, 256), full(256, 128), full(1, 128),
        ],
        out_specs=batched(128),
        out_shape=jax.ShapeDtypeStruct((_B, 128), f32),
    )(dense_pad, small_cat, large_cat,
      bw0p, row2(bb0), bw1, row2(bb1), bw2, row2(bb2),
      Up, Vp, dbp,
      tw0p, row2(tb0), tw1, row2(tb1), tw2, row2(tb2),
      tw3p, row2(tb3p))
    return out[:, :1]
